# baseline (speedup 1.0000x reference)
import functools

import jax
import jax.numpy as jnp
from jax import lax
from jax.experimental import pallas as pl
from jax.experimental.pallas import tpu as pltpu
from jax.experimental.pallas import tpu_sc as plsc

_CHUNK = 128


def _argmax_body(z_ref, idx_ref):
    z = z_ref[0]
    k = z.shape[1]
    m = jnp.max(z, axis=1, keepdims=True)
    iota = jax.lax.broadcasted_iota(jnp.int32, z.shape, 1)
    idx_ref[0, 0] = jnp.min(jnp.where(z == m, iota, k), axis=1)


def _make_sc_gather(n, d, nc, nw):
    n_per_w = n // nw
    n_chunks = n_per_w // _CHUNK
    mesh = plsc.VectorSubcoreMesh(core_axis_name="c", subcore_axis_name="s")

    @functools.partial(
        pl.kernel,
        out_type=jax.ShapeDtypeStruct((n, d), jnp.float32),
        mesh=mesh,
        scratch_types=[
            pltpu.VMEM((n_chunks, _CHUNK), jnp.int32),
            pltpu.VMEM((2, _CHUNK, d), jnp.float32),
            pltpu.SemaphoreType.DMA,
        ],
    )
    def sc_gather(w_hbm, idx_hbm, out_hbm, idx_v, rows_v, gsem):
        wid = lax.axis_index("s") * nc + lax.axis_index("c")
        base = wid * n_per_w
        pltpu.sync_copy(idx_hbm.at[wid], idx_v)
        prev = None
        for c in range(n_chunks):
            cp = pltpu.async_copy(w_hbm.at[idx_v.at[c]], rows_v.at[c % 2], gsem)
            if prev is not None:
                prev.wait()
                pltpu.sync_copy(
                    rows_v.at[(c - 1) % 2],
                    out_hbm.at[pl.ds(base + (c - 1) * _CHUNK, _CHUNK)],
                )
            prev = cp
        prev.wait()
        pltpu.sync_copy(
            rows_v.at[(n_chunks - 1) % 2],
            out_hbm.at[pl.ds(base + (n_chunks - 1) * _CHUNK, _CHUNK)],
        )

    return sc_gather


@jax.jit
def kernel(z_e_x, weight):
    b, k, h, w = z_e_x.shape
    d = weight.shape[1]
    hw = h * w
    n = b * hw
    z = jnp.transpose(z_e_x, (0, 2, 3, 1)).reshape(b, hw, k)
    idx = pl.pallas_call(
        _argmax_body,
        grid=(b,),
        in_specs=[pl.BlockSpec((1, hw, k), lambda i: (i, 0, 0))],
        out_specs=pl.BlockSpec((1, 1, hw), lambda i: (i, 0, 0)),
        out_shape=jax.ShapeDtypeStruct((b, 1, hw), jnp.int32),
    )(z)
    info = plsc.get_sparse_core_info()
    nc = info.num_cores
    nw = nc * info.num_subcores
    out = _make_sc_gather(n, d, nc, nw)(
        weight, idx.reshape(nw, n // (nw * _CHUNK), _CHUNK)
    )
    return out.reshape(b, h, w, d).transpose(0, 3, 1, 2)


# speedup vs baseline: 1.0081x; 1.0081x over previous
import functools

import jax
import jax.numpy as jnp
from jax import lax
from jax.experimental import pallas as pl
from jax.experimental.pallas import tpu as pltpu
from jax.experimental.pallas import tpu_sc as plsc

_CHUNK = 128
_NSPLIT = 4


def _make_argmax(b, hw, k, nb, s):
    def body(z_ref, idx_ref):
        z = z_ref[0]
        m = jnp.max(z, axis=1, keepdims=True)
        iota = jax.lax.broadcasted_iota(jnp.int32, z.shape, 1)
        idx_ref[0, 0] = jnp.min(jnp.where(z == m, iota, k), axis=1)

    return pl.pallas_call(
        body,
        grid=(nb,),
        in_specs=[pl.BlockSpec((1, hw, k), lambda i: (s * nb + i, 0, 0))],
        out_specs=pl.BlockSpec((1, 1, hw), lambda i: (i, 0, 0)),
        out_shape=jax.ShapeDtypeStruct((nb, 1, hw), jnp.int32),
    )


def _make_sc_gather(n, npart, d, nc, nw, s, with_out_type):
    n_per_w = npart // nw
    n_chunks = n_per_w // _CHUNK
    mesh = plsc.VectorSubcoreMesh(core_axis_name="c", subcore_axis_name="s")

    kwargs = {}
    if with_out_type:
        kwargs["out_type"] = jax.ShapeDtypeStruct((n, d), jnp.float32)

    @functools.partial(
        pl.kernel,
        mesh=mesh,
        scratch_types=[
            pltpu.VMEM((n_chunks, _CHUNK), jnp.int32),
            pltpu.VMEM((2, _CHUNK, d), jnp.float32),
            pltpu.SemaphoreType.DMA,
        ],
        **kwargs,
    )
    def sc_gather(w_hbm, idx_hbm, out_hbm, idx_v, rows_v, gsem):
        wid = lax.axis_index("s") * nc + lax.axis_index("c")
        base = s * npart + wid * n_per_w
        pltpu.sync_copy(idx_hbm.at[wid], idx_v)
        prev = None
        for c in range(n_chunks):
            cp = pltpu.async_copy(w_hbm.at[idx_v.at[c]], rows_v.at[c % 2], gsem)
            if prev is not None:
                prev.wait()
                pltpu.sync_copy(
                    rows_v.at[(c - 1) % 2],
                    out_hbm.at[pl.ds(base + (c - 1) * _CHUNK, _CHUNK)],
                )
            prev = cp
        prev.wait()
        pltpu.sync_copy(
            rows_v.at[(n_chunks - 1) % 2],
            out_hbm.at[pl.ds(base + (n_chunks - 1) * _CHUNK, _CHUNK)],
        )

    return sc_gather


@jax.jit
def kernel(z_e_x, weight):
    b, k, h, w = z_e_x.shape
    d = weight.shape[1]
    hw = h * w
    n = b * hw
    z = jnp.transpose(z_e_x, (0, 2, 3, 1)).reshape(b, hw, k)
    info = plsc.get_sparse_core_info()
    nc = info.num_cores
    nw = nc * info.num_subcores
    nb = b // _NSPLIT
    npart = n // _NSPLIT

    idxs = [_make_argmax(b, hw, k, nb, s)(z) for s in range(_NSPLIT)]
    idxs = [ix.reshape(nw, npart // (nw * _CHUNK), _CHUNK) for ix in idxs]

    out = _make_sc_gather(n, npart, d, nc, nw, 0, True)(weight, idxs[0])
    out_ref = jax.new_ref(out)
    for s in range(1, _NSPLIT):
        _make_sc_gather(n, npart, d, nc, nw, s, False)(weight, idxs[s], out_ref)
    out = out_ref[...]
    return out.reshape(b, h, w, d).transpose(0, 3, 1, 2)


# speedup vs baseline: 1.1669x; 1.1575x over previous
import functools

import jax
import jax.numpy as jnp
from jax.experimental import pallas as pl
from jax.experimental.pallas import tpu as pltpu


def _fused_body(w_ref, z_ref, o_ref):
    z = z_ref[0]
    k = z.shape[1]
    m = jnp.max(z, axis=1, keepdims=True)
    iota = jax.lax.broadcasted_iota(jnp.int32, z.shape, 1)
    idx = jnp.min(jnp.where(z == m, iota, k), axis=1, keepdims=True)
    onehot = (iota == idx).astype(w_ref.dtype)
    o_ref[0] = jax.lax.dot_general(
        onehot, w_ref[...], (((1,), (0,)), ((), ())),
        preferred_element_type=jnp.float32,
    )


@jax.jit
def kernel(z_e_x, weight):
    b, k, h, w = z_e_x.shape
    d = weight.shape[1]
    hw = h * w
    z = jnp.transpose(z_e_x, (0, 2, 3, 1)).reshape(b * 2, hw // 2, k)
    out = pl.pallas_call(
        _fused_body,
        grid=(b * 2,),
        in_specs=[
            pl.BlockSpec((k, d), lambda i: (0, 0)),
            pl.BlockSpec((1, hw // 2, k), lambda i: (i, 0, 0)),
        ],
        out_specs=pl.BlockSpec((1, hw // 2, d), lambda i: (i, 0, 0)),
        out_shape=jax.ShapeDtypeStruct((b * 2, hw // 2, d), jnp.float32),
    )(weight.astype(jnp.bfloat16), z)
    return out.reshape(b, h, w, d).transpose(0, 3, 1, 2)


# speedup vs baseline: 1.7466x; 1.4968x over previous
import functools

import jax
import jax.numpy as jnp
from jax.experimental import pallas as pl
from jax.experimental.pallas import tpu as pltpu


def _fused_body(w_ref, z_ref, o_ref):
    z = z_ref[0]
    k = z.shape[1]
    m = jnp.max(z, axis=1, keepdims=True)
    iota = jax.lax.broadcasted_iota(jnp.int32, z.shape, 1)
    idx = jnp.min(jnp.where(z == m, iota, k), axis=1, keepdims=True)
    onehot = (iota == idx).astype(w_ref.dtype)
    o_ref[0] = jax.lax.dot_general(
        onehot, w_ref[...], (((1,), (0,)), ((), ())),
        preferred_element_type=jnp.float32,
    )


@jax.jit
def kernel(z_e_x, weight):
    b, k, h, w = z_e_x.shape
    d = weight.shape[1]
    hw = h * w
    z = jnp.transpose(z_e_x, (0, 2, 3, 1)).reshape(b // 2, hw * 2, k)
    out = pl.pallas_call(
        _fused_body,
        grid=(b // 2,),
        in_specs=[
            pl.BlockSpec((k, d), lambda i: (0, 0)),
            pl.BlockSpec((1, hw * 2, k), lambda i: (i, 0, 0)),
        ],
        out_specs=pl.BlockSpec((1, hw * 2, d), lambda i: (i, 0, 0)),
        out_shape=jax.ShapeDtypeStruct((b // 2, hw * 2, d), jnp.float32),
    )(weight.astype(jnp.bfloat16), z)
    return out.reshape(b, h, w, d).transpose(0, 3, 1, 2)


# speedup vs baseline: 1.8023x; 1.0319x over previous
import functools

import jax
import jax.numpy as jnp
from jax.experimental import pallas as pl
from jax.experimental.pallas import tpu as pltpu


def _fused_body(w_ref, z_ref, o_ref):
    z = z_ref[0]
    k = z.shape[1]
    m = jnp.max(z, axis=1, keepdims=True)
    iota = jax.lax.broadcasted_iota(jnp.int32, z.shape, 1)
    idx = jnp.min(jnp.where(z == m, iota, k), axis=1, keepdims=True)
    onehot = (iota == idx).astype(w_ref.dtype)
    o_ref[0] = jax.lax.dot_general(
        onehot, w_ref[...], (((1,), (0,)), ((), ())),
        preferred_element_type=jnp.float32,
    )


@jax.jit
def kernel(z_e_x, weight):
    b, k, h, w = z_e_x.shape
    d = weight.shape[1]
    hw = h * w
    z = jnp.transpose(z_e_x, (0, 2, 3, 1)).reshape(b // 4, hw * 4, k)
    out = pl.pallas_call(
        _fused_body,
        grid=(b // 4,),
        in_specs=[
            pl.BlockSpec((k, d), lambda i: (0, 0)),
            pl.BlockSpec((1, hw * 4, k), lambda i: (i, 0, 0)),
        ],
        out_specs=pl.BlockSpec((1, hw * 4, d), lambda i: (i, 0, 0)),
        out_shape=jax.ShapeDtypeStruct((b // 4, hw * 4, d), jnp.float32),
    )(weight.astype(jnp.bfloat16), z)
    return out.reshape(b, h, w, d).transpose(0, 3, 1, 2)


# speedup vs baseline: 1.8628x; 1.0336x over previous
import functools

import jax
import jax.numpy as jnp
from jax.experimental import pallas as pl
from jax.experimental.pallas import tpu as pltpu


def _fused_body(w_ref, z_ref, o_ref):
    z = z_ref[0]
    k = z.shape[1]
    m = jnp.max(z, axis=1, keepdims=True)
    iota = jax.lax.broadcasted_iota(jnp.int32, z.shape, 1)
    idx = jnp.min(jnp.where(z == m, iota, k), axis=1, keepdims=True)
    onehot = (iota == idx).astype(jnp.bfloat16)
    o_ref[0] = jax.lax.dot_general(
        onehot, w_ref[...].astype(jnp.bfloat16), (((1,), (0,)), ((), ())),
        preferred_element_type=jnp.float32,
    )


@jax.jit
def kernel(z_e_x, weight):
    b, k, h, w = z_e_x.shape
    d = weight.shape[1]
    hw = h * w
    z = jnp.transpose(z_e_x, (0, 2, 3, 1)).reshape(b // 4, hw * 4, k)
    out = pl.pallas_call(
        _fused_body,
        grid=(b // 4,),
        in_specs=[
            pl.BlockSpec((k, d), lambda i: (0, 0)),
            pl.BlockSpec((1, hw * 4, k), lambda i: (i, 0, 0)),
        ],
        out_specs=pl.BlockSpec((1, hw * 4, d), lambda i: (i, 0, 0)),
        out_shape=jax.ShapeDtypeStruct((b // 4, hw * 4, d), jnp.float32),
    )(weight, z)
    return out.reshape(b, h, w, d).transpose(0, 3, 1, 2)
